# Initial kernel scaffold; baseline (speedup 1.0000x reference)
import jax
import jax.numpy as jnp
from jax.experimental import pallas as pl


def kernel(token_ids, segment_ids, input_ids, token_table, segment_table, position_table, ln_gamma, ln_beta):
    raise NotImplementedError("write your pallas kernel here")


# speedup vs baseline: 1.4507x; 1.4507x over previous
import functools

import jax
import jax.numpy as jnp
from jax import lax
from jax.experimental import pallas as pl
from jax.experimental.pallas import tpu as pltpu
from jax.experimental.pallas import tpu_sc as plsc

_NC = 2
_NS = 16
_NW = _NC * _NS

_GATHER_CHUNK = 64


def _sc_gather_rows(table, idx):
    n = idx.shape[0]
    d = table.shape[1]
    rpw = n // _NW
    ch = min(_GATHER_CHUNK, rpw)
    nch = rpw // ch
    assert rpw % ch == 0

    mesh = plsc.VectorSubcoreMesh(
        core_axis_name="c", subcore_axis_name="s",
        num_cores=_NC, num_subcores=_NS,
    )

    @functools.partial(
        pl.kernel,
        mesh=mesh,
        out_type=jax.ShapeDtypeStruct((n, d), jnp.float32),
        scratch_types=[
            pltpu.VMEM((ch,), jnp.int32),
            pltpu.VMEM((ch, d), jnp.float32),
            pltpu.SemaphoreType.DMA,
        ],
    )
    def k(table_hbm, idx_hbm, out_hbm, idx_v, rows_v, sem):
        wid = lax.axis_index("s") * _NC + lax.axis_index("c")
        base = wid * rpw
        for ci in range(nch):
            off = base + ci * ch
            pltpu.sync_copy(idx_hbm.at[pl.ds(off, ch)], idx_v)
            pltpu.async_copy(table_hbm.at[idx_v], rows_v, sem).wait()
            pltpu.sync_copy(rows_v, out_hbm.at[pl.ds(off, ch)])

    return k(table, idx)


_TC_BLOCK_ROWS = 256


def _tc_add_layernorm(tok, pos_table, seg_pad, seg_f, gamma2d, beta2d, eps):
    n, d = tok.shape
    s = pos_table.shape[0]
    br = _TC_BLOCK_ROWS
    assert n % br == 0 and s % br == 0
    nblk = n // br
    sblk = s // br

    def body(tok_ref, pos_ref, seg_ref, sid_ref, g_ref, b_ref, o_ref):
        s0 = seg_ref[0, :]
        sd = seg_ref[1, :] - s0
        x = tok_ref[...] + pos_ref[...] + s0[None, :] + sid_ref[...] * sd[None, :]
        mean = jnp.mean(x, axis=-1, keepdims=True)
        xc = x - mean
        var = jnp.mean(xc * xc, axis=-1, keepdims=True)
        inv = lax.rsqrt(var + eps)
        o_ref[...] = xc * inv * g_ref[...] + b_ref[...]

    return pl.pallas_call(
        body,
        grid=(nblk,),
        in_specs=[
            pl.BlockSpec((br, d), lambda i: (i, 0)),
            pl.BlockSpec((br, d), lambda i: (i % sblk, 0)),
            pl.BlockSpec((8, d), lambda i: (0, 0)),
            pl.BlockSpec((br, 1), lambda i: (i, 0)),
            pl.BlockSpec((1, d), lambda i: (0, 0)),
            pl.BlockSpec((1, d), lambda i: (0, 0)),
        ],
        out_specs=pl.BlockSpec((br, d), lambda i: (i, 0)),
        out_shape=jax.ShapeDtypeStruct((n, d), jnp.float32),
    )(tok, pos_table, seg_pad, seg_f, gamma2d, beta2d)


def kernel(token_ids, segment_ids, input_ids, token_table, segment_table,
           position_table, ln_gamma, ln_beta):
    b, s = input_ids.shape
    d = token_table.shape[1]
    n = b * s

    idx = token_ids.reshape(n).astype(jnp.int32)
    tok = _sc_gather_rows(token_table, idx)

    seg_pad = jnp.pad(segment_table, ((0, 8 - segment_table.shape[0]), (0, 0)))
    seg_f = segment_ids.reshape(n, 1).astype(jnp.float32)
    out = _tc_add_layernorm(
        tok, position_table, seg_pad, seg_f,
        ln_gamma.reshape(1, d), ln_beta.reshape(1, d), 1e-5,
    )
    return out.reshape(b, s, d)


# speedup vs baseline: 1.5061x; 1.0381x over previous
import functools

import jax
import jax.numpy as jnp
from jax import lax
from jax.experimental import pallas as pl
from jax.experimental.pallas import tpu as pltpu
from jax.experimental.pallas import tpu_sc as plsc

_NC = 2
_NS = 16
_NW = _NC * _NS

_GATHER_CHUNK = 64


def _sc_gather_rows(table, idx):
    n = idx.shape[0]
    d = table.shape[1]
    rpw = n // _NW
    ch = min(_GATHER_CHUNK, rpw)
    nch = rpw // ch
    assert rpw % ch == 0

    mesh = plsc.VectorSubcoreMesh(
        core_axis_name="c", subcore_axis_name="s",
        num_cores=_NC, num_subcores=_NS,
    )

    @functools.partial(
        pl.kernel,
        mesh=mesh,
        out_type=jax.ShapeDtypeStruct((n, d), jnp.float32),
        scratch_types=[
            pltpu.VMEM((rpw,), jnp.int32),
            pltpu.VMEM((ch, d), jnp.float32),
            pltpu.VMEM((ch, d), jnp.float32),
            pltpu.SemaphoreType.DMA,
            pltpu.SemaphoreType.DMA,
            pltpu.SemaphoreType.DMA,
            pltpu.SemaphoreType.DMA,
        ],
    )
    def k(table_hbm, idx_hbm, out_hbm, idx_v, rows0, rows1, g0, g1, s0, s1):
        wid = lax.axis_index("s") * _NC + lax.axis_index("c")
        base = wid * rpw
        pltpu.sync_copy(idx_hbm.at[pl.ds(base, rpw)], idx_v)
        rows = (rows0, rows1)
        gsem = (g0, g1)
        ssem = (s0, s1)

        def gather(ci):
            b = ci % 2
            return pltpu.async_copy(
                table_hbm.at[idx_v.at[pl.ds(ci * ch, ch)]], rows[b], gsem[b])

        def store(ci):
            b = ci % 2
            return pltpu.async_copy(
                rows[b], out_hbm.at[pl.ds(base + ci * ch, ch)], ssem[b])

        gcp = [None, None]
        scp = [None, None]
        gcp[0] = gather(0)
        for ci in range(nch):
            b = ci % 2
            nb = (ci + 1) % 2
            if ci + 1 < nch:
                if scp[nb] is not None:
                    scp[nb].wait()
                gcp[nb] = gather(ci + 1)
            gcp[b].wait()
            scp[b] = store(ci)
        for cp in scp:
            if cp is not None:
                cp.wait()

    return k(table, idx)


_TC_BLOCK_ROWS = 256


def _tc_add_layernorm(tok, pos_table, seg_pad, seg_f, gamma2d, beta2d, eps):
    n, d = tok.shape
    s = pos_table.shape[0]
    br = _TC_BLOCK_ROWS
    assert n % br == 0 and s % br == 0
    nblk = n // br
    sblk = s // br

    def body(tok_ref, pos_ref, seg_ref, sid_ref, g_ref, b_ref, o_ref):
        s0 = seg_ref[0, :]
        sd = seg_ref[1, :] - s0
        x = tok_ref[...] + pos_ref[...] + s0[None, :] + sid_ref[...] * sd[None, :]
        mean = jnp.mean(x, axis=-1, keepdims=True)
        xc = x - mean
        var = jnp.mean(xc * xc, axis=-1, keepdims=True)
        inv = lax.rsqrt(var + eps)
        o_ref[...] = xc * inv * g_ref[...] + b_ref[...]

    return pl.pallas_call(
        body,
        grid=(nblk,),
        in_specs=[
            pl.BlockSpec((br, d), lambda i: (i, 0)),
            pl.BlockSpec((br, d), lambda i: (i % sblk, 0)),
            pl.BlockSpec((8, d), lambda i: (0, 0)),
            pl.BlockSpec((br, 1), lambda i: (i, 0)),
            pl.BlockSpec((1, d), lambda i: (0, 0)),
            pl.BlockSpec((1, d), lambda i: (0, 0)),
        ],
        out_specs=pl.BlockSpec((br, d), lambda i: (i, 0)),
        out_shape=jax.ShapeDtypeStruct((n, d), jnp.float32),
    )(tok, pos_table, seg_pad, seg_f, gamma2d, beta2d)


def kernel(token_ids, segment_ids, input_ids, token_table, segment_table,
           position_table, ln_gamma, ln_beta):
    b, s = input_ids.shape
    d = token_table.shape[1]
    n = b * s

    idx = token_ids.reshape(n).astype(jnp.int32)
    tok = _sc_gather_rows(token_table, idx)

    seg_pad = jnp.pad(segment_table, ((0, 8 - segment_table.shape[0]), (0, 0)))
    seg_f = segment_ids.reshape(n, 1).astype(jnp.float32)
    out = _tc_add_layernorm(
        tok, position_table, seg_pad, seg_f,
        ln_gamma.reshape(1, d), ln_beta.reshape(1, d), 1e-5,
    )
    return out.reshape(b, s, d)
